# baseline (speedup 1.0000x reference)
from kernel_sc import kernel_sc


def kernel(activations, winners):
    return kernel_sc(activations, winners)


# speedup vs baseline: 5.1361x; 5.1361x over previous
import jax
import jax.numpy as jnp
from jax import lax
from jax.experimental import pallas as pl
from jax.experimental.pallas import tpu as pltpu
from jax.experimental.pallas import tpu_sc as plsc

K = 8
B = 128
C = 768
P = 256
RROWS = B * P
NC, NS = 2, 16
NW = NC * NS
RPW = RROWS // NW
BPW = B // NW
NG = C // 16
GPT = NG // NS
CH = 16
NCHK = RPW // CH
GRC = RPW * NG


def _body(a2_hbm, w6_hbm, wg_hbm, o2_hbm, thr_hbm,
          in0, in1, ot0, ot1, gbuf, thr_my, thr_v, wslab,
          is0, is1, os0, os1):
    cid = lax.axis_index("c")
    sid = lax.axis_index("s")
    wid = sid * NC + cid
    r0 = wid * RPW

    def in_slice(j):
        return a2_hbm.at[pl.ds(r0 + j * CH, CH), :]

    def out_slice(j):
        return o2_hbm.at[pl.ds(r0 + j * CH, CH), :]

    pltpu.make_async_copy(in_slice(0), in0, is0).start()
    pltpu.make_async_copy(in_slice(1), in1, is1).start()

    NEG = jnp.full((16,), -jnp.inf, jnp.float32)

    for gg in range(GPT):
        g = sid * GPT + gg
        pltpu.sync_copy(wg_hbm.at[g], gbuf)

        def ins(j, s):
            x = gbuf[j]
            out = []
            for k in range(K):
                hi = jnp.maximum(s[k], x)
                x = jnp.minimum(s[k], x)
                out.append(hi)
            return tuple(out)
        s = lax.fori_loop(0, B, ins, (NEG,) * K)
        thr_my[...] = s[K - 1]
        pltpu.sync_copy(thr_my, thr_hbm.at[(cid * NS + sid) * 8 + gg])

    plsc.subcore_barrier()
    pltpu.sync_copy(thr_hbm.at[pl.ds(cid * NS * 8, NS * 8)], thr_v)

    pltpu.sync_copy(w6_hbm.at[pl.ds(wid * (BPW * NG), BPW * NG)], wslab)

    HG = NG // 2

    def compute(j, ibuf, obuf):
        i = lax.div(j, jnp.int32(NCHK // BPW))
        for half in range(2):
            masks = tuple(
                jnp.where(
                    wslab[i * NG + half * HG + gg]
                    >= thr_v[8 * ((half * HG + gg) // GPT)
                             + (half * HG + gg) % GPT],
                    1.0, 0.0)
                for gg in range(HG))

            def rows(r, ms):
                for gg in range(HG):
                    gcol = 16 * (half * HG + gg)
                    obuf[r, pl.ds(gcol, 16)] = (
                        ibuf[r, pl.ds(gcol, 16)] * ms[gg])
                return ms
            lax.fori_loop(0, CH, rows, masks)

    def step(t, c):
        j0 = 2 * t
        j1 = 2 * t + 1

        pltpu.make_async_copy(in_slice(j0), in0, is0).wait()

        @pl.when(t > 0)
        def _():
            pltpu.make_async_copy(ot0, out_slice(j0 - 2), os0).wait()
        compute(j0, in0, ot0)
        pltpu.make_async_copy(ot0, out_slice(j0), os0).start()

        @pl.when(j0 + 2 < NCHK)
        def _():
            pltpu.make_async_copy(in_slice(j0 + 2), in0, is0).start()

        pltpu.make_async_copy(in_slice(j1), in1, is1).wait()

        @pl.when(t > 0)
        def _():
            pltpu.make_async_copy(ot1, out_slice(j1 - 2), os1).wait()
        compute(j1, in1, ot1)
        pltpu.make_async_copy(ot1, out_slice(j1), os1).start()

        @pl.when(j1 + 2 < NCHK)
        def _():
            pltpu.make_async_copy(in_slice(j1 + 2), in1, is1).start()
        return c

    lax.fori_loop(0, NCHK // 2, step, 0)
    pltpu.make_async_copy(ot0, out_slice(NCHK - 2), os0).wait()
    pltpu.make_async_copy(ot1, out_slice(NCHK - 1), os1).wait()


def kernel(activations, winners):
    a2 = activations.transpose(0, 2, 3, 1).reshape(RROWS, C)
    w6 = winners.reshape(B * NG, 16)
    wg = winners.reshape(B, NG, 16).transpose(1, 0, 2)
    mesh = plsc.VectorSubcoreMesh(core_axis_name="c", subcore_axis_name="s")
    out = pl.kernel(
        _body,
        out_type=(jax.ShapeDtypeStruct((RROWS, C), jnp.float32),
                  jax.ShapeDtypeStruct((NW * 8, 16), jnp.float32)),
        mesh=mesh,
        compiler_params=pltpu.CompilerParams(
            needs_layout_passes=False, use_tc_tiling_on_sc=True),
        scratch_types=[
            pltpu.VMEM((CH, C), jnp.float32),
            pltpu.VMEM((CH, C), jnp.float32),
            pltpu.VMEM((CH, C), jnp.float32),
            pltpu.VMEM((CH, C), jnp.float32),
            pltpu.VMEM((B, 16), jnp.float32),
            pltpu.VMEM((16,), jnp.float32),
            pltpu.VMEM((NS * 8, 16), jnp.float32),
            pltpu.VMEM((BPW * NG, 16), jnp.float32),
            pltpu.SemaphoreType.DMA,
            pltpu.SemaphoreType.DMA,
            pltpu.SemaphoreType.DMA,
            pltpu.SemaphoreType.DMA,
        ],
    )(a2, w6, wg)
    o = out[0].reshape(B, 16, 16, C).transpose(0, 3, 1, 2)
    return o
